# baseline (speedup 1.0000x reference)
import functools

import jax
import jax.numpy as jnp
from jax import lax
from jax.experimental import pallas as pl
from jax.experimental.pallas import tpu as pltpu
from jax.experimental.pallas import tpu_sc as plsc

_TM = 128
_NW = 32


def _matmul_body(te_ref, xs_ref, wgu_ref, wd_ref, gua_ref, gub_ref,
                 dat_ref, dbt_ref, out_ref, *, inter, nt):
    g = pl.program_id(0)
    n_used = te_ref[nt]

    @pl.when(g < n_used)
    def _work():
        x = xs_ref[...]
        gu = jnp.dot(x, wgu_ref[0], preferred_element_type=jnp.float32)

        cdim = (((1,), (1,)), ((), ()))
        xab = lax.dot_general(x, gua_ref[0], cdim,
                              preferred_element_type=jnp.float32)
        lgu = jnp.dot(xab, gub_ref[0], preferred_element_type=jnp.float32)
        gu = gu + lgu

        gate = gu[:, :inter]
        up = gu[:, inter:]
        act = jax.nn.silu(gate) * up

        y = jnp.dot(act, wd_ref[0], preferred_element_type=jnp.float32)
        xda = lax.dot_general(act, dat_ref[0], cdim,
                              preferred_element_type=jnp.float32)
        y = y + lax.dot_general(xda, dbt_ref[0], cdim,
                                preferred_element_type=jnp.float32)
        out_ref[...] = y


def kernel(hidden_states, topk_ids, topk_weights, gate_a, gate_b, up_a, up_b,
           down_a, down_b, weight_indices, seq_lens, lora_ranks, scalings,
           base_gate_up_weight, base_down_weight):
    T, H = hidden_states.shape
    E, _, I2 = base_gate_up_weight.shape
    inter = I2 // 2
    R = gate_a.shape[2]
    K = topk_ids.shape[1]
    TK = T * K
    tm = _TM
    nt = TK // tm + E
    P = nt * tm

    e_flat = topk_ids.reshape(-1).astype(jnp.int32)
    oh = (e_flat[:, None] == jnp.arange(E, dtype=jnp.int32)[None, :])
    csum = jnp.cumsum(oh.astype(jnp.int32), axis=0)
    counts = csum[-1]
    padded = ((counts + tm - 1) // tm) * tm
    cpad = jnp.cumsum(padded)
    pad_start = cpad - padded
    grp_start = jnp.cumsum(counts) - counts

    rank_flat = jnp.take_along_axis(csum, e_flat[:, None], axis=1)[:, 0] - 1
    pair_pos = (pad_start[e_flat] + rank_flat).reshape(T, K)

    order = jnp.argsort(e_flat).astype(jnp.int32)

    n_used = (cpad[E - 1] // tm).astype(jnp.int32)
    g_idx = jnp.arange(nt, dtype=jnp.int32)
    tile_expert = jnp.minimum(
        jnp.sum(g_idx[:, None] >= (cpad // tm)[None, :], axis=1,
                dtype=jnp.int32),
        E - 1).astype(jnp.int32)

    slot = jnp.arange(P, dtype=jnp.int32)
    ep = jnp.repeat(tile_expert, tm)
    srank = slot - pad_start[ep]
    valid = (srank >= 0) & (srank < counts[ep])
    src = jnp.clip(grp_start[ep] + srank, 0, TK - 1)
    tok_pad = jnp.where(valid, order[src] // K, slot % T)
    te_arr = jnp.concatenate([tile_expert, n_used[None]])

    adapter = weight_indices[0]
    s = scalings[adapter].astype(jnp.float32)
    gat = lax.dynamic_index_in_dim(gate_a, adapter, 0, False)
    uat = lax.dynamic_index_in_dim(up_a, adapter, 0, False)
    gbt = s * lax.dynamic_index_in_dim(gate_b, adapter, 0, False)
    ubt = s * lax.dynamic_index_in_dim(up_b, adapter, 0, False)
    dat = lax.dynamic_index_in_dim(down_a, adapter, 0, False)
    dbt = s * lax.dynamic_index_in_dim(down_b, adapter, 0, False)
    gua = jnp.concatenate([gat, uat], axis=1)
    gub = jnp.zeros((E, 2 * R, I2), jnp.float32)
    gub = lax.dynamic_update_slice(gub, gbt.transpose(0, 2, 1), (0, 0, 0))
    gub = lax.dynamic_update_slice(gub, ubt.transpose(0, 2, 1), (0, R, inter))

    x32 = hidden_states.astype(jnp.float32)
    mesh = plsc.VectorSubcoreMesh(core_axis_name="c", subcore_axis_name="s")

    rpw = P // _NW
    chunk = rpw // 2
    nchunk = rpw // chunk
    L = 16

    @functools.partial(
        pl.kernel, mesh=mesh,
        out_type=jax.ShapeDtypeStruct((P, H), jnp.float32),
        scratch_types=[pltpu.VMEM((2, chunk), jnp.int32),
                       pltpu.VMEM((2, chunk, H), jnp.float32),
                       pltpu.SemaphoreType.DMA,
                       pltpu.SemaphoreType.DMA])
    def gather_rows(x_hbm, idx_hbm, out_hbm, idx_v, rows_v, sem0, sem1):
        wid = lax.axis_index("s") * 2 + lax.axis_index("c")
        base = wid * rpw
        sems = (sem0, sem1)
        copies = [None, None]

        def issue(c):
            b = c % 2
            pltpu.sync_copy(idx_hbm.at[pl.ds(base + c * chunk, chunk)],
                            idx_v.at[b])
            copies[b] = pltpu.async_copy(x_hbm.at[idx_v.at[b]], rows_v.at[b],
                                         sems[b])

        issue(0)
        for c in range(nchunk):
            if c + 1 < nchunk:
                issue(c + 1)
            b = c % 2
            copies[b].wait()
            pltpu.sync_copy(rows_v.at[b],
                            out_hbm.at[pl.ds(base + c * chunk, chunk)])

    xs = gather_rows(x32, tok_pad)

    grid_spec = pltpu.PrefetchScalarGridSpec(
        num_scalar_prefetch=1,
        grid=(nt,),
        in_specs=[
            pl.BlockSpec((tm, H), lambda g, te: (g, 0)),
            pl.BlockSpec((1, H, I2), lambda g, te: (te[g], 0, 0)),
            pl.BlockSpec((1, inter, H), lambda g, te: (te[g], 0, 0)),
            pl.BlockSpec((1, 2 * R, H), lambda g, te: (te[g], 0, 0)),
            pl.BlockSpec((1, 2 * R, I2), lambda g, te: (te[g], 0, 0)),
            pl.BlockSpec((1, R, inter), lambda g, te: (te[g], 0, 0)),
            pl.BlockSpec((1, H, R), lambda g, te: (te[g], 0, 0)),
        ],
        out_specs=pl.BlockSpec((tm, H), lambda g, te: (g, 0)),
    )
    ys = pl.pallas_call(
        functools.partial(_matmul_body, inter=inter, nt=nt),
        grid_spec=grid_spec,
        out_shape=jax.ShapeDtypeStruct((P, H), jnp.float32),
    )(te_arr, xs, base_gate_up_weight, base_down_weight,
      gua, gub, dat, dbt)

    tpw = T // _NW
    wb = [jnp.broadcast_to(
        topk_weights[:, k].astype(jnp.float32)[:, None], (T, L))
        for k in range(K)]

    @functools.partial(
        pl.kernel, mesh=mesh,
        out_type=jax.ShapeDtypeStruct((T, H), jnp.float32),
        scratch_types=[pltpu.VMEM((2, tpw), jnp.int32),
                       pltpu.VMEM((tpw, H), jnp.float32),
                       pltpu.VMEM((tpw, H), jnp.float32),
                       pltpu.VMEM((tpw, L), jnp.float32),
                       pltpu.VMEM((tpw, L), jnp.float32),
                       pltpu.SemaphoreType.DMA,
                       pltpu.SemaphoreType.DMA])
    def combine_rows(ys_hbm, p0_hbm, p1_hbm, w0_hbm, w1_hbm, out_hbm,
                     idx_v, r0_v, r1_v, w0_v, w1_v, sem0, sem1):
        wid = lax.axis_index("s") * 2 + lax.axis_index("c")
        base = wid * tpw
        pltpu.sync_copy(p0_hbm.at[pl.ds(base, tpw)], idx_v.at[0])
        cp0 = pltpu.async_copy(ys_hbm.at[idx_v.at[0]], r0_v, sem0)
        pltpu.sync_copy(p1_hbm.at[pl.ds(base, tpw)], idx_v.at[1])
        cp1 = pltpu.async_copy(ys_hbm.at[idx_v.at[1]], r1_v, sem1)
        pltpu.sync_copy(w0_hbm.at[pl.ds(base, tpw)], w0_v)
        pltpu.sync_copy(w1_hbm.at[pl.ds(base, tpw)], w1_v)
        cp0.wait()
        cp1.wait()

        def body(i, carry):
            w0 = w0_v[i, :]
            w1 = w1_v[i, :]
            for j in range(H // L):
                sl = pl.ds(j * L, L)
                r0_v[i, sl] = w0 * r0_v[i, sl] + w1 * r1_v[i, sl]
            return carry

        lax.fori_loop(0, tpw, body, 0)
        pltpu.sync_copy(r0_v, out_hbm.at[pl.ds(base, tpw)])

    out = combine_rows(ys, pair_pos[:, 0], pair_pos[:, 1], wb[0], wb[1])
    return out.astype(hidden_states.dtype)


# speedup vs baseline: 1.0444x; 1.0444x over previous
import functools

import jax
import jax.numpy as jnp
from jax import lax
from jax.experimental import pallas as pl
from jax.experimental.pallas import tpu as pltpu
from jax.experimental.pallas import tpu_sc as plsc

_TM = 256
_NW = 32


def _matmul_body(te_ref, xs_ref, wgu_ref, wd_ref, gua_ref, gub_ref,
                 dat_ref, dbt_ref, out_ref, *, inter, nt):
    g = pl.program_id(0)
    n_used = te_ref[nt]

    @pl.when(g < n_used)
    def _work():
        x = xs_ref[...]
        gu = jnp.dot(x, wgu_ref[0], preferred_element_type=jnp.float32)

        cdim = (((1,), (1,)), ((), ()))
        xab = lax.dot_general(x, gua_ref[0], cdim,
                              preferred_element_type=jnp.float32)
        lgu = jnp.dot(xab, gub_ref[0], preferred_element_type=jnp.float32)
        gu = gu + lgu

        gate = gu[:, :inter]
        up = gu[:, inter:]
        act = jax.nn.silu(gate) * up

        y = jnp.dot(act, wd_ref[0], preferred_element_type=jnp.float32)
        xda = lax.dot_general(act, dat_ref[0], cdim,
                              preferred_element_type=jnp.float32)
        y = y + lax.dot_general(xda, dbt_ref[0], cdim,
                                preferred_element_type=jnp.float32)
        out_ref[...] = y


def kernel(hidden_states, topk_ids, topk_weights, gate_a, gate_b, up_a, up_b,
           down_a, down_b, weight_indices, seq_lens, lora_ranks, scalings,
           base_gate_up_weight, base_down_weight):
    T, H = hidden_states.shape
    E, _, I2 = base_gate_up_weight.shape
    inter = I2 // 2
    R = gate_a.shape[2]
    K = topk_ids.shape[1]
    TK = T * K
    tm = _TM
    nt = TK // tm + E
    P = nt * tm

    e_flat = topk_ids.reshape(-1).astype(jnp.int32)
    oh = (e_flat[:, None] == jnp.arange(E, dtype=jnp.int32)[None, :])
    csum = jnp.cumsum(oh.astype(jnp.int32), axis=0)
    counts = csum[-1]
    padded = ((counts + tm - 1) // tm) * tm
    cpad = jnp.cumsum(padded)
    pad_start = cpad - padded
    grp_start = jnp.cumsum(counts) - counts

    rank_flat = jnp.take_along_axis(csum, e_flat[:, None], axis=1)[:, 0] - 1
    pair_pos = (pad_start[e_flat] + rank_flat).reshape(T, K)

    order = jnp.argsort(e_flat).astype(jnp.int32)

    n_used = (cpad[E - 1] // tm).astype(jnp.int32)
    g_idx = jnp.arange(nt, dtype=jnp.int32)
    tile_expert = jnp.minimum(
        jnp.sum(g_idx[:, None] >= (cpad // tm)[None, :], axis=1,
                dtype=jnp.int32),
        E - 1).astype(jnp.int32)

    slot = jnp.arange(P, dtype=jnp.int32)
    ep = jnp.repeat(tile_expert, tm)
    srank = slot - pad_start[ep]
    valid = (srank >= 0) & (srank < counts[ep])
    src = jnp.clip(grp_start[ep] + srank, 0, TK - 1)
    tok_pad = jnp.where(valid, order[src] // K, slot % T)
    te_arr = jnp.concatenate([tile_expert, n_used[None]])

    adapter = weight_indices[0]
    s = scalings[adapter].astype(jnp.float32)
    gat = lax.dynamic_index_in_dim(gate_a, adapter, 0, False)
    uat = lax.dynamic_index_in_dim(up_a, adapter, 0, False)
    gbt = s * lax.dynamic_index_in_dim(gate_b, adapter, 0, False)
    ubt = s * lax.dynamic_index_in_dim(up_b, adapter, 0, False)
    dat = lax.dynamic_index_in_dim(down_a, adapter, 0, False)
    dbt = s * lax.dynamic_index_in_dim(down_b, adapter, 0, False)
    gua = jnp.concatenate([gat, uat], axis=1)
    gub = jnp.zeros((E, 2 * R, I2), jnp.float32)
    gub = lax.dynamic_update_slice(gub, gbt.transpose(0, 2, 1), (0, 0, 0))
    gub = lax.dynamic_update_slice(gub, ubt.transpose(0, 2, 1), (0, R, inter))

    x32 = hidden_states.astype(jnp.float32)
    mesh = plsc.VectorSubcoreMesh(core_axis_name="c", subcore_axis_name="s")

    rpw = P // _NW
    chunk = 64
    nchunk = rpw // chunk
    L = 16

    @functools.partial(
        pl.kernel, mesh=mesh,
        out_type=jax.ShapeDtypeStruct((P, H), jnp.float32),
        scratch_types=[pltpu.VMEM((2, chunk), jnp.int32),
                       pltpu.VMEM((2, chunk, H), jnp.float32),
                       pltpu.SemaphoreType.DMA,
                       pltpu.SemaphoreType.DMA])
    def gather_rows(x_hbm, idx_hbm, out_hbm, idx_v, rows_v, sem0, sem1):
        wid = lax.axis_index("s") * 2 + lax.axis_index("c")
        base = wid * rpw
        sems = (sem0, sem1)
        copies = [None, None]

        def issue(c):
            b = c % 2
            pltpu.sync_copy(idx_hbm.at[pl.ds(base + c * chunk, chunk)],
                            idx_v.at[b])
            copies[b] = pltpu.async_copy(x_hbm.at[idx_v.at[b]], rows_v.at[b],
                                         sems[b])

        issue(0)
        for c in range(nchunk):
            if c + 1 < nchunk:
                issue(c + 1)
            b = c % 2
            copies[b].wait()
            pltpu.sync_copy(rows_v.at[b],
                            out_hbm.at[pl.ds(base + c * chunk, chunk)])

    xs = gather_rows(x32, tok_pad)

    grid_spec = pltpu.PrefetchScalarGridSpec(
        num_scalar_prefetch=1,
        grid=(nt,),
        in_specs=[
            pl.BlockSpec((tm, H), lambda g, te: (g, 0)),
            pl.BlockSpec((1, H, I2), lambda g, te: (te[g], 0, 0)),
            pl.BlockSpec((1, inter, H), lambda g, te: (te[g], 0, 0)),
            pl.BlockSpec((1, 2 * R, H), lambda g, te: (te[g], 0, 0)),
            pl.BlockSpec((1, 2 * R, I2), lambda g, te: (te[g], 0, 0)),
            pl.BlockSpec((1, R, inter), lambda g, te: (te[g], 0, 0)),
            pl.BlockSpec((1, H, R), lambda g, te: (te[g], 0, 0)),
        ],
        out_specs=pl.BlockSpec((tm, H), lambda g, te: (g, 0)),
    )
    ys = pl.pallas_call(
        functools.partial(_matmul_body, inter=inter, nt=nt),
        grid_spec=grid_spec,
        out_shape=jax.ShapeDtypeStruct((P, H), jnp.float32),
    )(te_arr, xs, base_gate_up_weight, base_down_weight,
      gua, gub, dat, dbt)

    tpw = T // _NW
    wb = [jnp.broadcast_to(
        topk_weights[:, k].astype(jnp.float32)[:, None], (T, L))
        for k in range(K)]

    @functools.partial(
        pl.kernel, mesh=mesh,
        out_type=jax.ShapeDtypeStruct((T, H), jnp.float32),
        scratch_types=[pltpu.VMEM((2, tpw), jnp.int32),
                       pltpu.VMEM((tpw, H), jnp.float32),
                       pltpu.VMEM((tpw, H), jnp.float32),
                       pltpu.VMEM((tpw, L), jnp.float32),
                       pltpu.VMEM((tpw, L), jnp.float32),
                       pltpu.SemaphoreType.DMA,
                       pltpu.SemaphoreType.DMA])
    def combine_rows(ys_hbm, p0_hbm, p1_hbm, w0_hbm, w1_hbm, out_hbm,
                     idx_v, r0_v, r1_v, w0_v, w1_v, sem0, sem1):
        wid = lax.axis_index("s") * 2 + lax.axis_index("c")
        base = wid * tpw
        pltpu.sync_copy(p0_hbm.at[pl.ds(base, tpw)], idx_v.at[0])
        cp0 = pltpu.async_copy(ys_hbm.at[idx_v.at[0]], r0_v, sem0)
        pltpu.sync_copy(p1_hbm.at[pl.ds(base, tpw)], idx_v.at[1])
        cp1 = pltpu.async_copy(ys_hbm.at[idx_v.at[1]], r1_v, sem1)
        pltpu.sync_copy(w0_hbm.at[pl.ds(base, tpw)], w0_v)
        pltpu.sync_copy(w1_hbm.at[pl.ds(base, tpw)], w1_v)
        cp0.wait()
        cp1.wait()

        def body(i, carry):
            w0 = w0_v[i, :]
            w1 = w1_v[i, :]
            for j in range(H // L):
                sl = pl.ds(j * L, L)
                r0_v[i, sl] = w0 * r0_v[i, sl] + w1 * r1_v[i, sl]
            return carry

        lax.fori_loop(0, tpw, body, 0)
        pltpu.sync_copy(r0_v, out_hbm.at[pl.ds(base, tpw)])

    out = combine_rows(ys, pair_pos[:, 0], pair_pos[:, 1], wb[0], wb[1])
    return out.astype(hidden_states.dtype)
